# baseline (speedup 1.0000x reference)
import functools

import jax
import jax.numpy as jnp
from jax.experimental import pallas as pl
from jax.experimental.pallas import tpu as pltpu

D_MODEL = 1024
N_EXP = 16
D_EXPERT = 256
N_SHARED = 1
K_FFN = 2
N_ROUTED = N_EXP - N_SHARED
S = 2048


def _moe_kernel(x_ref, sel_ref, es_ref, k_ref, v_ref,
                out_ref, idx_ref, w_ref):
    e = pl.program_id(0)

    @pl.when(e == 0)
    def _routing():
        logits = jnp.dot(sel_ref[...], es_ref[...].T,
                         preferred_element_type=jnp.float32)
        aff = jax.nn.sigmoid(logits)
        ids = jax.lax.broadcasted_iota(jnp.int32, (S, N_EXP), 1)
        neg = jnp.where(ids < N_ROUTED, aff, -jnp.inf)
        m1 = jnp.max(neg, axis=1, keepdims=True)
        i1 = jnp.min(jnp.where(neg == m1, ids, N_EXP), axis=1, keepdims=True)
        neg2 = jnp.where(ids == i1, -jnp.inf, neg)
        m2 = jnp.max(neg2, axis=1, keepdims=True)
        i2 = jnp.min(jnp.where(neg2 == m2, ids, N_EXP), axis=1, keepdims=True)
        shared = jnp.full((S, 1), N_ROUTED, dtype=jnp.int32)
        idx_ref[...] = jnp.concatenate([i1, i2, shared], axis=1)
        selmask = (ids == i1) | (ids == i2) | (ids >= N_ROUTED)
        w_ref[...] = jnp.where(selmask, aff, 0.0)

    onehot = (jax.lax.broadcasted_iota(jnp.int32, (N_EXP, 1), 0) == e
              ).astype(jnp.float32)
    wcol = jnp.dot(w_ref[...], onehot,
                   preferred_element_type=jnp.float32)
    h = jnp.dot(x_ref[...], k_ref[0], preferred_element_type=jnp.float32)
    h = h * jax.nn.sigmoid(h)
    hw = (h * wcol).astype(jnp.bfloat16)
    y = jnp.dot(hw, v_ref[0], preferred_element_type=jnp.float32)

    @pl.when(e == 0)
    def _init():
        out_ref[...] = y

    @pl.when(e > 0)
    def _acc():
        out_ref[...] += y


@jax.jit
def kernel(token_stream, selection_input, keys_w, values_w, expert_sel):
    x = token_stream.reshape(S, D_MODEL).astype(jnp.bfloat16)
    sel = selection_input.reshape(S, D_MODEL)
    keys_w = keys_w.astype(jnp.bfloat16)
    values_w = values_w.astype(jnp.bfloat16)

    out, sel_idx = pl.pallas_call(
        _moe_kernel,
        grid=(N_EXP,),
        in_specs=[
            pl.BlockSpec((S, D_MODEL), lambda e: (0, 0)),
            pl.BlockSpec((S, D_MODEL), lambda e: (0, 0)),
            pl.BlockSpec((N_EXP, D_MODEL), lambda e: (0, 0)),
            pl.BlockSpec((1, D_MODEL, D_EXPERT), lambda e: (e, 0, 0)),
            pl.BlockSpec((1, D_EXPERT, D_MODEL), lambda e: (e, 0, 0)),
        ],
        out_specs=[
            pl.BlockSpec((S, D_MODEL), lambda e: (0, 0)),
            pl.BlockSpec((S, 3), lambda e: (0, 0)),
        ],
        out_shape=[
            jax.ShapeDtypeStruct((S, D_MODEL), jnp.float32),
            jax.ShapeDtypeStruct((S, 3), jnp.int32),
        ],
        scratch_shapes=[pltpu.VMEM((S, N_EXP), jnp.float32)],
        compiler_params=pltpu.CompilerParams(
            dimension_semantics=("arbitrary",),
        ),
    )(x, sel, expert_sel, keys_w, values_w)

    return out.reshape(1, S, D_MODEL), sel_idx.reshape(1, S, 3)


# speedup vs baseline: 1.4888x; 1.4888x over previous
import jax
import jax.numpy as jnp
from jax.experimental import pallas as pl
from jax.experimental.pallas import tpu as pltpu

D_MODEL = 1024
N_EXP = 16
D_EXPERT = 256
N_SHARED = 1
K_FFN = 2
N_ROUTED = N_EXP - N_SHARED
S = 2048
BLK = 256


def _moe_kernel(x_ref, sel_ref, est_ref, k_ref, v_ref, out_ref, idx_ref):
    logits = jnp.dot(sel_ref[...], est_ref[...],
                     preferred_element_type=jnp.float32)
    aff = jax.nn.sigmoid(logits)
    ids = jax.lax.broadcasted_iota(jnp.int32, (BLK, N_EXP), 1)
    neg = jnp.where(ids < N_ROUTED, aff, -jnp.inf)
    m1 = jnp.max(neg, axis=1, keepdims=True)
    i1 = jnp.min(jnp.where(neg == m1, ids, N_EXP), axis=1, keepdims=True)
    neg2 = jnp.where(ids == i1, -jnp.inf, neg)
    m2 = jnp.max(neg2, axis=1, keepdims=True)
    i2 = jnp.min(jnp.where(neg2 == m2, ids, N_EXP), axis=1, keepdims=True)
    shared = jnp.full((BLK, 1), N_ROUTED, dtype=jnp.int32)
    idx_ref[...] = jnp.concatenate([i1, i2, shared], axis=1)
    selmask = (ids == i1) | (ids == i2) | (ids >= N_ROUTED)
    w = jnp.where(selmask, aff, 0.0)

    x = x_ref[...]
    acc = jnp.zeros((BLK, D_MODEL), dtype=jnp.float32)
    for e in range(N_EXP):
        h = jnp.dot(x, k_ref[e], preferred_element_type=jnp.float32)
        h = h * jax.nn.sigmoid(h)
        hw = (h * w[:, e:e + 1]).astype(jnp.bfloat16)
        acc = acc + jnp.dot(hw, v_ref[e], preferred_element_type=jnp.float32)
    out_ref[...] = acc


@jax.jit
def kernel(token_stream, selection_input, keys_w, values_w, expert_sel):
    x = token_stream.reshape(S, D_MODEL).astype(jnp.bfloat16)
    sel = selection_input.reshape(S, D_MODEL)
    est = expert_sel.T
    keys_w = keys_w.astype(jnp.bfloat16)
    values_w = values_w.astype(jnp.bfloat16)

    out, sel_idx = pl.pallas_call(
        _moe_kernel,
        grid=(S // BLK,),
        in_specs=[
            pl.BlockSpec((BLK, D_MODEL), lambda t: (t, 0)),
            pl.BlockSpec((BLK, D_MODEL), lambda t: (t, 0)),
            pl.BlockSpec((D_MODEL, N_EXP), lambda t: (0, 0)),
            pl.BlockSpec((N_EXP, D_MODEL, D_EXPERT), lambda t: (0, 0, 0)),
            pl.BlockSpec((N_EXP, D_EXPERT, D_MODEL), lambda t: (0, 0, 0)),
        ],
        out_specs=[
            pl.BlockSpec((BLK, D_MODEL), lambda t: (t, 0)),
            pl.BlockSpec((BLK, 3), lambda t: (t, 0)),
        ],
        out_shape=[
            jax.ShapeDtypeStruct((S, D_MODEL), jnp.float32),
            jax.ShapeDtypeStruct((S, 3), jnp.int32),
        ],
        compiler_params=pltpu.CompilerParams(
            dimension_semantics=("arbitrary",),
        ),
    )(x, sel, est, keys_w, values_w)

    return out.reshape(1, S, D_MODEL), sel_idx.reshape(1, S, 3)
